# baseline (speedup 1.0000x reference)
import functools

import jax
import jax.numpy as jnp
from jax import lax
from jax.experimental import pallas as pl
from jax.experimental.pallas import tpu as pltpu
from jax.experimental.pallas import tpu_sc as plsc

N_TOKENS = 16384
D_MODEL = 4096
NUM_EXPERTS = 64
TOP_K = 8

_NC = 2
_NS = 16
_NW = _NC * _NS
_L = 16

_ROWS_PER_W = N_TOKENS // _NW


def _mm_body(x_ref, w_ref, o_ref):
    o_ref[...] = jnp.dot(x_ref[...], w_ref[...],
                         preferred_element_type=jnp.float32)


def _matmul_chunk(inputs, w_gate, chunk, n_chunks, bm=512):
    n, d = inputs.shape
    e = w_gate.shape[1]
    rows = n // n_chunks
    steps = rows // bm
    base = chunk * steps
    return pl.pallas_call(
        _mm_body,
        grid=(steps,),
        in_specs=[
            pl.BlockSpec((bm, d), lambda i: (base + i, 0)),
            pl.BlockSpec(memory_space=pltpu.VMEM),
        ],
        out_specs=pl.BlockSpec((bm, e), lambda i: (i, 0)),
        out_shape=jax.ShapeDtypeStruct((rows, e), jnp.float32),
    )(inputs, w_gate)


_GATHER_DNUMS = lax.GatherDimensionNumbers(
    offset_dims=(), collapsed_slice_dims=(0,), start_index_map=(0,))


def _lane_bcast(x, i):
    idx = jnp.full((_L, 1), i, jnp.int32)
    return lax.gather(x, idx, _GATHER_DNUMS, (1,),
                      mode=lax.GatherScatterMode.PROMISE_IN_BOUNDS)


_NQ = 2


def _route_body(rw_hbm, out_hbm, rw_v, out_v, sems, *, rows_per_w):
    wid = lax.axis_index("s") * _NC + lax.axis_index("c")
    base = wid * rows_per_w
    q = rows_per_w // _NQ

    lane = lax.iota(jnp.int32, _L)
    lo8 = lane < 8

    def sort_desc(k, v):
        r = plsc.sort_key_val(k, v, descending=True)
        return r[0], r[1]

    def merge_desc(a, b):
        ck = jnp.where(lo8, a[0], lax.rev(b[0], (0,)))
        cv = jnp.where(lo8, a[1], lax.rev(b[1], (0,)))
        return sort_desc(ck, cv)

    idx = [lane + 16 * j for j in range(4)]
    zeros = jnp.zeros((_L,), jnp.float32)
    ones = jnp.ones((_L,), jnp.float32)

    def compute_q(s):
        @plsc.parallel_loop(0, q, unroll=6)
        def row(r):
            v0 = rw_v[s, r, pl.ds(0, _L)]
            v1 = rw_v[s, r, pl.ds(16, _L)]
            v2 = rw_v[s, r, pl.ds(32, _L)]
            v3 = rw_v[s, r, pl.ds(48, _L)]
            m01 = merge_desc(sort_desc(v0, idx[0]), sort_desc(v1, idx[1]))
            m23 = merge_desc(sort_desc(v2, idx[2]), sort_desc(v3, idx[3]))
            mk, mi = merge_desc(m01, m23)
            rowmax = _lane_bcast(mk, 0)
            em = jnp.where(lo8, jnp.exp(mk - rowmax), 0.0)
            den = jnp.broadcast_to(jnp.sum(em), (_L,))
            g = em * (ones / den)
            out_v[s, r, pl.ds(0, _L)] = zeros
            out_v[s, r, pl.ds(16, _L)] = zeros
            out_v[s, r, pl.ds(32, _L)] = zeros
            out_v[s, r, pl.ds(48, _L)] = zeros
            rr = jnp.broadcast_to(r, (_L,)).astype(jnp.int32)
            plsc.store_scatter(out_v.at[s], [rr, mi], g, mask=lo8)

    def in_copy(h):
        return pltpu.async_copy(
            rw_hbm.at[pl.ds(base + h * q, q)], rw_v.at[h % 2],
            sems.at[h % 2])

    def out_copy(h):
        return pltpu.async_copy(
            out_v.at[h % 2], out_hbm.at[pl.ds(base + h * q, q)],
            sems.at[2 + h % 2])

    in_copies = [in_copy(0), in_copy(1)]
    out_copies = []
    for h in range(_NQ):
        in_copies[h].wait()
        if h >= 2:
            out_copies[h - 2].wait()
        compute_q(h % 2)
        out_copies.append(out_copy(h))
        if h + 2 < _NQ:
            in_copies.append(in_copy(h + 2))
    for c in out_copies[-2:]:
        c.wait()


@functools.lru_cache(maxsize=None)
def _make_route(n_rows):
    rows_per_w = n_rows // _NW
    return pl.kernel(
        functools.partial(_route_body, rows_per_w=rows_per_w),
        out_type=jax.ShapeDtypeStruct((n_rows, NUM_EXPERTS), jnp.float32),
        mesh=plsc.VectorSubcoreMesh(core_axis_name="c", subcore_axis_name="s"),
        scratch_types=[
            pltpu.VMEM((2, rows_per_w // _NQ, NUM_EXPERTS), jnp.float32),
            pltpu.VMEM((2, rows_per_w // _NQ, NUM_EXPERTS), jnp.float32),
            pltpu.SemaphoreType.DMA((4,)),
        ],
        compiler_params=pltpu.CompilerParams(needs_layout_passes=False),
    )


_MM_BM = 512
_MM_NBUF = 4


def _mm_ring_body(x_hbm, w_ref, o_ref, buf, sem):
    n = x_hbm.shape[0]
    nsteps = n // _MM_BM

    def dma(i, slot):
        return pltpu.make_async_copy(
            x_hbm.at[pl.ds(i * _MM_BM, _MM_BM)], buf.at[slot], sem.at[slot])

    for i in range(_MM_NBUF):
        dma(i, i).start()

    def step(i, _):
        slot = lax.rem(i, _MM_NBUF)
        dma(i, slot).wait()
        o_ref[pl.ds(i * _MM_BM, _MM_BM), :] = jnp.dot(
            buf[slot], w_ref[...], preferred_element_type=jnp.float32)
        nxt = i + _MM_NBUF

        @pl.when(nxt < nsteps)
        def _():
            dma(nxt, slot).start()

        return _

    lax.fori_loop(0, nsteps, step, None)


def _matmul_ring(inputs, w_gate):
    n, d = inputs.shape
    e = w_gate.shape[1]
    return pl.pallas_call(
        _mm_ring_body,
        in_specs=[
            pl.BlockSpec(memory_space=pltpu.HBM),
            pl.BlockSpec(memory_space=pltpu.VMEM),
        ],
        out_specs=pl.BlockSpec(memory_space=pltpu.VMEM),
        out_shape=jax.ShapeDtypeStruct((n, e), jnp.float32),
        scratch_shapes=[
            pltpu.VMEM((_MM_NBUF, _MM_BM, d), jnp.float32),
            pltpu.SemaphoreType.DMA((_MM_NBUF,)),
        ],
    )(inputs, w_gate)


@jax.jit
def kernel(inputs, w_gate):
    rw = _matmul_chunk(inputs, w_gate, 0, 1)
    return _make_route(inputs.shape[0])(rw)


# speedup vs baseline: 1.0078x; 1.0078x over previous
import functools

import jax
import jax.numpy as jnp
from jax import lax
from jax.experimental import pallas as pl
from jax.experimental.pallas import tpu as pltpu
from jax.experimental.pallas import tpu_sc as plsc

N_TOKENS = 16384
D_MODEL = 4096
NUM_EXPERTS = 64
TOP_K = 8

_NC = 2
_NS = 16
_NW = _NC * _NS
_L = 16

def _mm_body(x_ref, w_ref, o_ref):
    o_ref[...] = jnp.dot(x_ref[...], w_ref[...],
                         preferred_element_type=jnp.float32)


def _matmul(inputs, w_gate, bm=512):
    n, d = inputs.shape
    e = w_gate.shape[1]
    return pl.pallas_call(
        _mm_body,
        grid=(n // bm,),
        in_specs=[
            pl.BlockSpec((bm, d), lambda i: (i, 0)),
            pl.BlockSpec(memory_space=pltpu.VMEM),
        ],
        out_specs=pl.BlockSpec((bm, e), lambda i: (i, 0)),
        out_shape=jax.ShapeDtypeStruct((n, e), jnp.float32),
    )(inputs, w_gate)


_GATHER_DNUMS = lax.GatherDimensionNumbers(
    offset_dims=(), collapsed_slice_dims=(0,), start_index_map=(0,))


def _lane_bcast(x, i):
    idx = jnp.full((_L, 1), i, jnp.int32)
    return lax.gather(x, idx, _GATHER_DNUMS, (1,),
                      mode=lax.GatherScatterMode.PROMISE_IN_BOUNDS)


_NQ = 2


def _route_body(rw_hbm, out_hbm, rw_v, out_v, sems, *, rows_per_w):
    wid = lax.axis_index("s") * _NC + lax.axis_index("c")
    base = wid * rows_per_w
    q = rows_per_w // _NQ

    lane = lax.iota(jnp.int32, _L)
    lo8 = lane < 8

    def sort_desc(k, v):
        r = plsc.sort_key_val(k, v, descending=True)
        return r[0], r[1]

    def merge_desc(a, b):
        ck = jnp.where(lo8, a[0], lax.rev(b[0], (0,)))
        cv = jnp.where(lo8, a[1], lax.rev(b[1], (0,)))
        return sort_desc(ck, cv)

    idx = [lane + 16 * j for j in range(4)]
    zeros = jnp.zeros((_L,), jnp.float32)
    ones = jnp.ones((_L,), jnp.float32)

    def compute_q(s):
        @plsc.parallel_loop(0, q, unroll=4)
        def row(r):
            v0 = rw_v[s, r, pl.ds(0, _L)]
            v1 = rw_v[s, r, pl.ds(16, _L)]
            v2 = rw_v[s, r, pl.ds(32, _L)]
            v3 = rw_v[s, r, pl.ds(48, _L)]
            m01 = merge_desc(sort_desc(v0, idx[0]), sort_desc(v1, idx[1]))
            m23 = merge_desc(sort_desc(v2, idx[2]), sort_desc(v3, idx[3]))
            mk, mi = merge_desc(m01, m23)
            rowmax = _lane_bcast(mk, 0)
            em = jnp.where(lo8, jnp.exp(mk - rowmax), 0.0)
            den = jnp.broadcast_to(jnp.sum(em), (_L,))
            g = em * (ones / den)
            out_v[s, r, pl.ds(0, _L)] = zeros
            out_v[s, r, pl.ds(16, _L)] = zeros
            out_v[s, r, pl.ds(32, _L)] = zeros
            out_v[s, r, pl.ds(48, _L)] = zeros
            rr = jnp.broadcast_to(r, (_L,)).astype(jnp.int32)
            plsc.store_scatter(out_v.at[s], [rr, mi], g, mask=lo8)

    def in_copy(h):
        return pltpu.async_copy(
            rw_hbm.at[pl.ds(base + h * q, q)], rw_v.at[h % 2],
            sems.at[h % 2])

    def out_copy(h):
        return pltpu.async_copy(
            out_v.at[h % 2], out_hbm.at[pl.ds(base + h * q, q)],
            sems.at[2 + h % 2])

    in_copies = [in_copy(0), in_copy(1)]
    out_copies = []
    for h in range(_NQ):
        in_copies[h].wait()
        if h >= 2:
            out_copies[h - 2].wait()
        compute_q(h % 2)
        out_copies.append(out_copy(h))
        if h + 2 < _NQ:
            in_copies.append(in_copy(h + 2))
    for c in out_copies[-2:]:
        c.wait()


@functools.lru_cache(maxsize=None)
def _make_route(n_rows):
    rows_per_w = n_rows // _NW
    return pl.kernel(
        functools.partial(_route_body, rows_per_w=rows_per_w),
        out_type=jax.ShapeDtypeStruct((n_rows, NUM_EXPERTS), jnp.float32),
        mesh=plsc.VectorSubcoreMesh(core_axis_name="c", subcore_axis_name="s"),
        scratch_types=[
            pltpu.VMEM((2, rows_per_w // _NQ, NUM_EXPERTS), jnp.float32),
            pltpu.VMEM((2, rows_per_w // _NQ, NUM_EXPERTS), jnp.float32),
            pltpu.SemaphoreType.DMA((4,)),
        ],
        compiler_params=pltpu.CompilerParams(needs_layout_passes=False),
    )


@jax.jit
def kernel(inputs, w_gate):
    rw = _matmul(inputs, w_gate)
    return _make_route(inputs.shape[0])(rw)
